# baseline (speedup 1.0000x reference)
import functools

import jax
import jax.numpy as jnp
from jax import lax
from jax.experimental import pallas as pl
from jax.experimental.pallas import tpu as pltpu
from jax.experimental.pallas import tpu_sc as plsc

_VOCAB = 100000
_HIDDEN = 1024
_BATCH = 4
_SEQ = 2048
_B = _BATCH * _SEQ
_NC = 2
_NS = 16
_NW = _NC * _NS
_BPW = _B // _NW
_WPR = _SEQ // _BPW
_K = 8
_NCHUNK = _BPW // _K
_NBUF = 14
_DEPTH = 10

_mesh = plsc.VectorSubcoreMesh(core_axis_name="c", subcore_axis_name="s")


@functools.partial(
    pl.kernel,
    mesh=_mesh,
    out_type=jax.ShapeDtypeStruct((_B, _HIDDEN), jnp.float32),
    scratch_types=(
        [pltpu.VMEM((_BPW,), jnp.int32)]
        + [pltpu.VMEM((_K, _HIDDEN), jnp.float32) for _ in range(_NBUF)]
        + [pltpu.SemaphoreType.DMA for _ in range(2 * _NBUF)]
    ),
)
def _gather_kernel(idx_hbm, table_hbm, out_hbm, idx_v, *rest):
    bufs = rest[:_NBUF]
    gsems = rest[_NBUF:2 * _NBUF]
    osems = rest[2 * _NBUF:]
    wid = lax.axis_index("s") * _NC + lax.axis_index("c")
    base = wid * _BPW
    row = wid // _WPR
    col = (wid % _WPR) * _BPW
    pltpu.sync_copy(idx_hbm.at[row, pl.ds(col, _BPW)], idx_v)

    def gather(c):
        r = c % _NBUF
        pltpu.async_copy(
            table_hbm.at[idx_v.at[pl.ds(c * _K, _K)]], bufs[r], gsems[r])

    def out_slice(c):
        return out_hbm.at[pl.ds(base + c * _K, _K)]

    for c in range(_DEPTH):
        gather(c)
    for c in range(_NCHUNK):
        r = c % _NBUF
        pltpu.make_async_copy(
            table_hbm.at[idx_v.at[pl.ds(c * _K, _K)]], bufs[r], gsems[r]).wait()
        pltpu.async_copy(bufs[r], out_slice(c), osems[r])
        n = c + _DEPTH
        if n < _NCHUNK:
            nr = n % _NBUF
            prev = n - _NBUF
            if prev >= 0:
                pltpu.make_async_copy(bufs[nr], out_slice(prev), osems[nr]).wait()
            gather(n)
    for c in range(_NCHUNK - _NBUF, _NCHUNK):
        if c >= 0:
            r = c % _NBUF
            pltpu.make_async_copy(bufs[r], out_slice(c), osems[r]).wait()


def kernel(idx, weight):
    batch, seq = idx.shape
    out = _gather_kernel(idx, weight)
    return out.reshape(batch, seq, weight.shape[1])


# speedup vs baseline: 1.0289x; 1.0289x over previous
import functools

import jax
import jax.numpy as jnp
from jax import lax
from jax.experimental import pallas as pl
from jax.experimental.pallas import tpu as pltpu
from jax.experimental.pallas import tpu_sc as plsc

_VOCAB = 100000
_HIDDEN = 1024
_BATCH = 4
_SEQ = 2048
_B = _BATCH * _SEQ
_NC = 2
_NS = 16
_NW = _NC * _NS
_BPW = _B // _NW
_WPR = _SEQ // _BPW
_K = 16
_NCHUNK = _BPW // _K
_NBUF = 7
_DEPTH = 6

_mesh = plsc.VectorSubcoreMesh(core_axis_name="c", subcore_axis_name="s")


@functools.partial(
    pl.kernel,
    mesh=_mesh,
    out_type=jax.ShapeDtypeStruct((_B, _HIDDEN), jnp.float32),
    scratch_types=(
        [pltpu.VMEM((_BPW,), jnp.int32)]
        + [pltpu.VMEM((_K, _HIDDEN), jnp.float32) for _ in range(_NBUF)]
        + [pltpu.SemaphoreType.DMA for _ in range(2 * _NBUF)]
    ),
)
def _gather_kernel(idx_hbm, table_hbm, out_hbm, idx_v, *rest):
    bufs = rest[:_NBUF]
    gsems = rest[_NBUF:2 * _NBUF]
    osems = rest[2 * _NBUF:]
    wid = lax.axis_index("s") * _NC + lax.axis_index("c")
    base = wid * _BPW
    row = wid // _WPR
    col = (wid % _WPR) * _BPW
    pltpu.sync_copy(idx_hbm.at[row, pl.ds(col, _BPW)], idx_v)

    def gather(c):
        r = c % _NBUF
        pltpu.async_copy(
            table_hbm.at[idx_v.at[pl.ds(c * _K, _K)]], bufs[r], gsems[r])

    def out_slice(c):
        return out_hbm.at[pl.ds(base + c * _K, _K)]

    for c in range(_DEPTH):
        gather(c)
    for c in range(_NCHUNK):
        r = c % _NBUF
        pltpu.make_async_copy(
            table_hbm.at[idx_v.at[pl.ds(c * _K, _K)]], bufs[r], gsems[r]).wait()
        pltpu.async_copy(bufs[r], out_slice(c), osems[r])
        n = c + _DEPTH
        if n < _NCHUNK:
            nr = n % _NBUF
            prev = n - _NBUF
            if prev >= 0:
                pltpu.make_async_copy(bufs[nr], out_slice(prev), osems[nr]).wait()
            gather(n)
    for c in range(_NCHUNK - _NBUF, _NCHUNK):
        if c >= 0:
            r = c % _NBUF
            pltpu.make_async_copy(bufs[r], out_slice(c), osems[r]).wait()


def kernel(idx, weight):
    batch, seq = idx.shape
    out = _gather_kernel(idx, weight)
    return out.reshape(batch, seq, weight.shape[1])


# speedup vs baseline: 1.0326x; 1.0036x over previous
import functools

import jax
import jax.numpy as jnp
from jax import lax
from jax.experimental import pallas as pl
from jax.experimental.pallas import tpu as pltpu
from jax.experimental.pallas import tpu_sc as plsc

_VOCAB = 100000
_HIDDEN = 1024
_BATCH = 4
_SEQ = 2048
_B = _BATCH * _SEQ
_NC = 2
_NS = 16
_NW = _NC * _NS
_BPW = _B // _NW
_WPR = _SEQ // _BPW
_K = 16
_NCHUNK = _BPW // _K
_NBUF = 7
_DEPTH = 5

_mesh = plsc.VectorSubcoreMesh(core_axis_name="c", subcore_axis_name="s")


@functools.partial(
    pl.kernel,
    mesh=_mesh,
    out_type=jax.ShapeDtypeStruct((_B, _HIDDEN), jnp.float32),
    scratch_types=(
        [pltpu.VMEM((_BPW,), jnp.int32)]
        + [pltpu.VMEM((_K, _HIDDEN), jnp.float32) for _ in range(_NBUF)]
        + [pltpu.SemaphoreType.DMA for _ in range(2 * _NBUF)]
    ),
)
def _gather_kernel(idx_hbm, table_hbm, out_hbm, idx_v, *rest):
    bufs = rest[:_NBUF]
    gsems = rest[_NBUF:2 * _NBUF]
    osems = rest[2 * _NBUF:]
    wid = lax.axis_index("s") * _NC + lax.axis_index("c")
    base = wid * _BPW
    row = wid // _WPR
    col = (wid % _WPR) * _BPW
    pltpu.sync_copy(idx_hbm.at[row, pl.ds(col, _BPW)], idx_v)

    def gather(c):
        r = c % _NBUF
        pltpu.async_copy(
            table_hbm.at[idx_v.at[pl.ds(c * _K, _K)]], bufs[r], gsems[r])

    def out_slice(c):
        return out_hbm.at[pl.ds(base + c * _K, _K)]

    for c in range(_DEPTH):
        gather(c)
    for c in range(_NCHUNK):
        r = c % _NBUF
        pltpu.make_async_copy(
            table_hbm.at[idx_v.at[pl.ds(c * _K, _K)]], bufs[r], gsems[r]).wait()
        pltpu.async_copy(bufs[r], out_slice(c), osems[r])
        n = c + _DEPTH
        if n < _NCHUNK:
            nr = n % _NBUF
            prev = n - _NBUF
            if prev >= 0:
                pltpu.make_async_copy(bufs[nr], out_slice(prev), osems[nr]).wait()
            gather(n)
    for c in range(_NCHUNK - _NBUF, _NCHUNK):
        if c >= 0:
            r = c % _NBUF
            pltpu.make_async_copy(bufs[r], out_slice(c), osems[r]).wait()


def kernel(idx, weight):
    batch, seq = idx.shape
    out = _gather_kernel(idx, weight)
    return out.reshape(batch, seq, weight.shape[1])
